# baseline (speedup 1.0000x reference)
import functools

import jax
import jax.numpy as jnp
from jax import lax
from jax.experimental import pallas as pl
from jax.experimental.pallas import tpu as pltpu
from jax.experimental.pallas import tpu_sc as plsc

F = 768
S = 2048
B = 2
N = B * S
E = 8
C3 = 2304
TE = N // E
NC = 2
NS = 16
NW = NC * NS


def _sc_multi_gather(tasks):
    mesh = plsc.VectorSubcoreMesh(core_axis_name="c", subcore_axis_name="s")
    per_ws = []
    for n_out, width, dtype, chunk in tasks:
        per_w = n_out // NW
        assert per_w % chunk == 0 and chunk <= 128 and chunk % 8 == 0
        per_ws.append(per_w)

    scratch = []
    for (n_out, width, dtype, chunk), per_w in zip(tasks, per_ws):
        scratch.append(pltpu.VMEM((per_w,), jnp.int32))
        scratch.append(pltpu.VMEM((chunk, width), dtype))
    scratch.append(pltpu.SemaphoreType.DMA)

    @functools.partial(
        pl.kernel,
        out_type=[jax.ShapeDtypeStruct((n_out, width), dtype)
                  for n_out, width, dtype, chunk in tasks],
        mesh=mesh,
        scratch_types=scratch,
    )
    def k(*refs):
        nt = len(tasks)
        tables = refs[:nt]
        idxs = refs[nt:2 * nt]
        outs = refs[2 * nt:3 * nt]
        sv = refs[3 * nt:]
        sem = sv[-1]
        wid = lax.axis_index("s") * NC + lax.axis_index("c")
        for t, (n_out, width, dtype, chunk) in enumerate(tasks):
            per_w = per_ws[t]
            idx_v, buf = sv[2 * t], sv[2 * t + 1]
            base = wid * per_w
            pltpu.sync_copy(idxs[t].at[pl.ds(base, per_w)], idx_v)
            for i in range(per_w // chunk):
                idx_c = idx_v if per_w == chunk else \
                    idx_v.at[pl.ds(i * chunk, chunk)]
                pltpu.async_copy(tables[t].at[idx_c], buf, sem).wait()
                pltpu.sync_copy(
                    buf, outs[t].at[pl.ds(base + i * chunk, chunk)])

    def call(*table_and_idx):
        res = k(*table_and_idx)
        return res if len(tasks) > 1 else res[0]

    return call


def _transpose_to_tokens(x):
    Sb = 256

    def body(x_ref, o_ref):
        o_ref[...] = x_ref[0].T

    return pl.pallas_call(
        body,
        grid=(B, S // Sb),
        in_specs=[pl.BlockSpec((1, F, Sb), lambda b, s: (b, 0, s))],
        out_specs=pl.BlockSpec((Sb, F), lambda b, s: (b * (S // Sb) + s, 0)),
        out_shape=jax.ShapeDtypeStruct((N, F), jnp.float32),
    )(x)


def _transpose_to_features(x):
    Sb = 256

    def body(x_ref, o_ref):
        o_ref[...] = x_ref[...].astype(jnp.float32).T[None]

    return pl.pallas_call(
        body,
        grid=(B, S // Sb),
        in_specs=[pl.BlockSpec((Sb, F), lambda b, s: (b * (S // Sb) + s, 0))],
        out_specs=pl.BlockSpec((1, F, Sb), lambda b, s: (b, 0, s)),
        out_shape=jax.ShapeDtypeStruct((B, F, S), jnp.float32),
    )(x)


def _moe0_matmul(xs, w0):
    def body(x_ref, w_ref, dep_ref, ss_ref):
        res = []
        for g in range(4):
            xg = x_ref[:, g * 192:(g + 1) * 192]
            res.append(jnp.dot(
                xg, w_ref[g], preferred_element_type=jnp.float32,
                precision=lax.Precision.DEFAULT))
        dep_ref[:, :576] = res[0]
        dep_ref[:, 576:] = res[1][:, :192]
        ss_ref[:, :384] = res[1][:, 192:].astype(jnp.bfloat16)
        ss_ref[:, 384:960] = res[2].astype(jnp.bfloat16)
        ss_ref[:, 960:] = res[3].astype(jnp.bfloat16)

    return pl.pallas_call(
        body,
        grid=(E,),
        in_specs=[pl.BlockSpec((TE, F), lambda e: (e, 0)),
                  pl.BlockSpec((4, 192, 576), lambda e: (e, 0, 0))],
        out_specs=[pl.BlockSpec((TE, F), lambda e: (e, 0)),
                   pl.BlockSpec((TE, 1536), lambda e: (e, 0))],
        out_shape=[jax.ShapeDtypeStruct((N, F), jnp.float32),
                   jax.ShapeDtypeStruct((N, 1536), jnp.bfloat16)],
    )(xs, w0)


def _norm_block(s0, s1, shift):
    s0r = jnp.maximum(s0, 0.0)
    x = s0r * s0r * s0r * s1 + shift
    x = x - jnp.mean(x, axis=1, keepdims=True)
    ssq = jnp.sum(x * x, axis=1, keepdims=True)
    return x * lax.rsqrt(ssq * (1.0 / F))


def _cum_norm_conv_norm(d_dep, d_ss, w1t):
    Rb = 256
    per_batch = S // Rb

    def body(dep_ref, sc_ref, sh_ref, w_ref, o_ref, carry_ref, tail_ref):
        i = pl.program_id(0)

        @pl.when(i % per_batch == 0)
        def _():
            carry_ref[...] = jnp.zeros_like(carry_ref)
            tail_ref[...] = jnp.zeros_like(tail_ref)

        r = lax.broadcasted_iota(jnp.int32, (Rb, Rb), 0)
        c = lax.broadcasted_iota(jnp.int32, (Rb, Rb), 1)
        ltri = (r >= c).astype(jnp.float32)
        cum = jnp.dot(ltri, dep_ref[...], preferred_element_type=jnp.float32,
                      precision=lax.Precision.DEFAULT) + carry_ref[...]
        carry_ref[...] = cum[Rb - 1:Rb, :]
        srow = (i % per_batch) * Rb + lax.broadcasted_iota(
            jnp.int32, (Rb, 1), 0)
        s0 = cum / (srow + 1).astype(jnp.float32)
        x1 = _norm_block(s0, sc_ref[...].astype(jnp.float32),
                         sh_ref[...].astype(jnp.float32))

        ext = jnp.concatenate([tail_ref[...], x1], axis=0)
        tail_ref[...] = x1[Rb - 2:, :]
        shifted = [ext[0:Rb], ext[1:Rb + 1], x1]
        cols = []
        for g in range(4):
            acc = None
            for k in range(3):
                xg = shifted[k][:, g * 192:(g + 1) * 192]
                wgk = w_ref[k, g * 576:(g + 1) * 576, :]
                pk = lax.dot_general(
                    xg, wgk, (((1,), (1,)), ((), ())),
                    preferred_element_type=jnp.float32,
                    precision=lax.Precision.DEFAULT)
                acc = pk if acc is None else acc + pk
            cols.append(acc)
        conv = jnp.concatenate(cols, axis=1)
        o_ref[...] = _norm_block(conv[:, :768], conv[:, 768:1536],
                                 conv[:, 1536:]).astype(jnp.bfloat16)

    return pl.pallas_call(
        body,
        grid=(N // Rb,),
        in_specs=[pl.BlockSpec((Rb, F), lambda i: (i, 0)),
                  pl.BlockSpec((Rb, F), lambda i: (i, 0)),
                  pl.BlockSpec((Rb, F), lambda i: (i, 1)),
                  pl.BlockSpec((3, C3, 192), lambda i: (0, 0, 0))],
        out_specs=pl.BlockSpec((Rb, F), lambda i: (i, 0)),
        out_shape=jax.ShapeDtypeStruct((N, F), jnp.bfloat16),
        scratch_shapes=[pltpu.VMEM((1, F), jnp.float32),
                        pltpu.VMEM((2, F), jnp.float32)],
    )(d_dep, d_ss, d_ss, w1t)


def _moe2_matmul(xs2, wsm, gcol):
    def body(x_ref, w_ref, g_ref, o_ref):
        x = x_ref[...].astype(jnp.float32)
        gc = g_ref[...]
        for g in range(4):
            xg = x * (gc == g).astype(jnp.float32)
            o_ref[:, g * 192:(g + 1) * 192] = jnp.dot(
                xg, w_ref[0, :, :192], preferred_element_type=jnp.float32,
                precision=lax.Precision.DEFAULT).astype(jnp.bfloat16)

    return pl.pallas_call(
        body,
        grid=(E,),
        in_specs=[pl.BlockSpec((TE, F), lambda e: (e, 0)),
                  pl.BlockSpec((1, F, 256), lambda e: (e, 0, 0)),
                  pl.BlockSpec((1, F), lambda e: (0, 0))],
        out_specs=pl.BlockSpec((TE, F), lambda e: (e, 0)),
        out_shape=jax.ShapeDtypeStruct((N, F), jnp.bfloat16),
    )(xs2, wsm, gcol)


def _as_f32rows(x):
    r, w = x.shape
    return lax.bitcast_convert_type(x.reshape(r, w // 2, 2), jnp.float32)


def _as_bf16rows(x, w):
    return lax.bitcast_convert_type(x, jnp.bfloat16).reshape(x.shape[0], w)

def _routing_indices():
    rkey = jax.random.key(1234)
    ka, kb = jax.random.split(rkey)
    idxs = []
    for key in (ka, kb):
        perm = jax.random.permutation(key, N).astype(jnp.int32)
        j = jnp.arange(N, dtype=jnp.int32)
        t_of_j = (j % TE) * E + j // TE
        src = perm[t_of_j]
        oslot = (perm % E) * TE + perm // E
        idxs.append((src, oslot))
    return idxs


def kernel(inp, w0, w1, w2, fs0, fs2):
    (src0, oslot0), (src2, oslot2) = _routing_indices()
    fs2_inv = jnp.argsort(fs2).astype(jnp.int32)
    idx_shuf0 = (jnp.repeat(jnp.arange(B, dtype=jnp.int32) * F, F)
                 + jnp.tile(fs0.astype(jnp.int32), B))
    qidx = (jnp.arange(E, dtype=jnp.int32)[:, None] * F
            + fs2_inv[None, :]).reshape(-1)
    gcol = (fs2_inv // 192).reshape(1, F)
    w1t = jnp.transpose(w1, (2, 0, 1))

    shuf = _sc_multi_gather([(B * F, S, jnp.float32, 48)])(
        inp.reshape(B * F, S), idx_shuf0)
    xtm = _transpose_to_tokens(shuf.reshape(B, F, S))
    w2p = jnp.pad(w2.reshape(E * F, 192), ((0, 0), (0, 64)))
    xs0, wsm = _sc_multi_gather([(N, F, jnp.float32, 128),
                                 (E * F, 256, jnp.float32, 96)])(
        xtm, w2p, src0, qidx)
    y0_dep, y0_ss = _moe0_matmul(xs0, w0)
    d_dep, d_ssv = _sc_multi_gather([(N, F, jnp.float32, 64),
                                     (N, F, jnp.float32, 64)])(
        y0_dep, _as_f32rows(y0_ss), oslot0, oslot0)
    x2 = _cum_norm_conv_norm(d_dep, _as_bf16rows(d_ssv, 1536), w1t)
    xs2v = _sc_multi_gather([(N, F // 2, jnp.float32, 128)])(
        _as_f32rows(x2), src2)
    y2 = _moe2_matmul(_as_bf16rows(xs2v, F), wsm.reshape(E, F, 256), gcol)
    outv = _sc_multi_gather([(N, F // 2, jnp.float32, 128)])(
        _as_f32rows(y2), oslot2)
    return _transpose_to_features(_as_bf16rows(outv, F))


# speedup vs baseline: 3.0639x; 3.0639x over previous
import functools

import jax
import jax.numpy as jnp
from jax import lax
from jax.experimental import pallas as pl
from jax.experimental.pallas import tpu as pltpu
from jax.experimental.pallas import tpu_sc as plsc

F = 768
S = 2048
B = 2
N = B * S
E = 8
C3 = 2304
TE = N // E
NC = 2
NS = 16
NW = NC * NS


def _sc_multi_gather(tasks):
    mesh = plsc.VectorSubcoreMesh(core_axis_name="c", subcore_axis_name="s")
    per_ws = []
    for n_out, width, dtype, chunk in tasks:
        per_w = n_out // NW
        assert per_w % chunk == 0 and chunk <= 128 and chunk % 8 == 0
        per_ws.append(per_w)

    scratch = []
    for (n_out, width, dtype, chunk), per_w in zip(tasks, per_ws):
        scratch.append(pltpu.VMEM((per_w,), jnp.int32))
        scratch.append(pltpu.VMEM((chunk, width), dtype))
    scratch.append(pltpu.SemaphoreType.DMA)

    @functools.partial(
        pl.kernel,
        out_type=[jax.ShapeDtypeStruct((n_out, width), dtype)
                  for n_out, width, dtype, chunk in tasks],
        mesh=mesh,
        scratch_types=scratch,
    )
    def k(*refs):
        nt = len(tasks)
        tables = refs[:nt]
        idxs = refs[nt:2 * nt]
        outs = refs[2 * nt:3 * nt]
        sv = refs[3 * nt:]
        sem = sv[-1]
        wid = lax.axis_index("s") * NC + lax.axis_index("c")
        for t, (n_out, width, dtype, chunk) in enumerate(tasks):
            per_w = per_ws[t]
            idx_v, buf = sv[2 * t], sv[2 * t + 1]
            base = wid * per_w
            pltpu.sync_copy(idxs[t].at[pl.ds(base, per_w)], idx_v)
            for i in range(per_w // chunk):
                idx_c = idx_v if per_w == chunk else \
                    idx_v.at[pl.ds(i * chunk, chunk)]
                pltpu.async_copy(tables[t].at[idx_c], buf, sem).wait()
                pltpu.sync_copy(
                    buf, outs[t].at[pl.ds(base + i * chunk, chunk)])

    def call(*table_and_idx):
        res = k(*table_and_idx)
        return res if len(tasks) > 1 else res[0]

    return call


def _transpose_to_tokens(x):
    Sb = 256

    def body(x_ref, o_ref):
        o_ref[...] = x_ref[0].T

    return pl.pallas_call(
        body,
        grid=(B, S // Sb),
        in_specs=[pl.BlockSpec((1, F, Sb), lambda b, s: (b, 0, s))],
        out_specs=pl.BlockSpec((Sb, F), lambda b, s: (b * (S // Sb) + s, 0)),
        out_shape=jax.ShapeDtypeStruct((N, F), jnp.float32),
    )(x)


def _transpose_to_features(x):
    Sb = 256

    def body(x_ref, o_ref):
        o_ref[...] = _unpack_half(x_ref[...]).T[None]

    return pl.pallas_call(
        body,
        grid=(B, S // Sb),
        in_specs=[pl.BlockSpec((Sb, 384), lambda b, s: (b * (S // Sb) + s, 0))],
        out_specs=pl.BlockSpec((1, F, Sb), lambda b, s: (b, 0, s)),
        out_shape=jax.ShapeDtypeStruct((B, F, S), jnp.float32),
    )(x)


def _moe0_matmul(xs, w0):
    def body(x_ref, w_ref, dep_ref, ss_ref):
        res = []
        for g in range(4):
            xg = x_ref[:, g * 192:(g + 1) * 192]
            res.append(jnp.dot(
                xg, w_ref[g], preferred_element_type=jnp.float32,
                precision=lax.Precision.DEFAULT))
        dep_ref[:, :576] = res[0]
        dep_ref[:, 576:] = res[1][:, :192]
        ss_ref[:, :384] = _pack16(res[1][:, 192:], res[2][:, :384])
        sh_lo = jnp.concatenate([res[2][:, 384:], res[3][:, :192]], axis=1)
        ss_ref[:, 384:] = _pack16(sh_lo, res[3][:, 192:])

    return pl.pallas_call(
        body,
        grid=(E,),
        in_specs=[pl.BlockSpec((TE, F), lambda e: (e, 0)),
                  pl.BlockSpec((4, 192, 576), lambda e: (e, 0, 0))],
        out_specs=[pl.BlockSpec((TE, F), lambda e: (e, 0)),
                   pl.BlockSpec((TE, F), lambda e: (e, 0))],
        out_shape=[jax.ShapeDtypeStruct((N, F), jnp.float32),
                   jax.ShapeDtypeStruct((N, F), jnp.int32)],
    )(xs, w0)


def _pack16(a, b):
    ai = lax.bitcast_convert_type(a, jnp.int32)
    bi = lax.bitcast_convert_type(b, jnp.int32)
    ra = (ai + 0x7FFF + ((ai >> 16) & 1)) >> 16
    rb = (bi + 0x7FFF + ((bi >> 16) & 1)) >> 16
    return (ra & 0xFFFF) | (rb << 16)


def _unpack16(p):
    a = lax.bitcast_convert_type(p << 16, jnp.float32)
    b = lax.bitcast_convert_type(p & jnp.int32(-65536), jnp.float32)
    return a, b


def _pack_half(x):
    return _pack16(x[:, :384], x[:, 384:])


def _unpack_half(p):
    a, b = _unpack16(p)
    return jnp.concatenate([a, b], axis=1)


def _norm_block(s0, s1, shift):
    s0r = jnp.maximum(s0, 0.0)
    x = s0r * s0r * s0r * s1 + shift
    x = x - jnp.mean(x, axis=1, keepdims=True)
    ssq = jnp.sum(x * x, axis=1, keepdims=True)
    return x * lax.rsqrt(ssq * (1.0 / F))


def _cum_norm_conv_norm(d_dep, d_ss, w1t):
    Rb = 256
    per_batch = S // Rb

    def body(dep_ref, sc_ref, sh_ref, w_ref, o_ref, carry_ref, tail_ref):
        i = pl.program_id(0)

        @pl.when(i % per_batch == 0)
        def _():
            carry_ref[...] = jnp.zeros_like(carry_ref)
            tail_ref[...] = jnp.zeros_like(tail_ref)

        r = lax.broadcasted_iota(jnp.int32, (Rb, Rb), 0)
        c = lax.broadcasted_iota(jnp.int32, (Rb, Rb), 1)
        ltri = (r >= c).astype(jnp.float32)
        cum = jnp.dot(ltri, dep_ref[...], preferred_element_type=jnp.float32,
                      precision=lax.Precision.DEFAULT) + carry_ref[...]
        carry_ref[...] = cum[Rb - 1:Rb, :]
        srow = (i % per_batch) * Rb + lax.broadcasted_iota(
            jnp.int32, (Rb, 1), 0)
        s0 = cum / (srow + 1).astype(jnp.float32)
        x1 = _norm_block(s0, _unpack_half(sc_ref[...]),
                         _unpack_half(sh_ref[...]))

        ext = jnp.concatenate([tail_ref[...], x1], axis=0)
        tail_ref[...] = x1[Rb - 2:, :]
        shifted = [ext[0:Rb], ext[1:Rb + 1], x1]
        cols = []
        for g in range(4):
            acc = None
            for k in range(3):
                xg = shifted[k][:, g * 192:(g + 1) * 192]
                wgk = w_ref[k, g * 576:(g + 1) * 576, :]
                pk = lax.dot_general(
                    xg, wgk, (((1,), (1,)), ((), ())),
                    preferred_element_type=jnp.float32,
                    precision=lax.Precision.DEFAULT)
                acc = pk if acc is None else acc + pk
            cols.append(acc)
        conv = jnp.concatenate(cols, axis=1)
        o_ref[...] = _pack_half(_norm_block(
            conv[:, :768], conv[:, 768:1536], conv[:, 1536:]))

    return pl.pallas_call(
        body,
        grid=(N // Rb,),
        in_specs=[pl.BlockSpec((Rb, F), lambda i: (i, 0)),
                  pl.BlockSpec((Rb, 384), lambda i: (i, 0)),
                  pl.BlockSpec((Rb, 384), lambda i: (i, 1)),
                  pl.BlockSpec((3, C3, 192), lambda i: (0, 0, 0))],
        out_specs=pl.BlockSpec((Rb, 384), lambda i: (i, 0)),
        out_shape=jax.ShapeDtypeStruct((N, 384), jnp.int32),
        scratch_shapes=[pltpu.VMEM((1, F), jnp.float32),
                        pltpu.VMEM((2, F), jnp.float32)],
    )(d_dep, d_ss, d_ss, w1t)


def _moe2_matmul(xs2, wsm, gcol):
    def body(x_ref, w_ref, g_ref, o_ref):
        x = _unpack_half(x_ref[...])
        gc = g_ref[...]
        ys = []
        for g in range(4):
            xg = x * (gc == g).astype(jnp.float32)
            ys.append(jnp.dot(
                xg, w_ref[0, :, :192], preferred_element_type=jnp.float32,
                precision=lax.Precision.DEFAULT))
        y = jnp.concatenate(ys, axis=1)
        o_ref[...] = _pack_half(y)

    return pl.pallas_call(
        body,
        grid=(E,),
        in_specs=[pl.BlockSpec((TE, 384), lambda e: (e, 0)),
                  pl.BlockSpec((1, F, 256), lambda e: (e, 0, 0)),
                  pl.BlockSpec((1, F), lambda e: (0, 0))],
        out_specs=pl.BlockSpec((TE, 384), lambda e: (e, 0)),
        out_shape=jax.ShapeDtypeStruct((N, 384), jnp.int32),
    )(xs2, wsm, gcol)


def _as_f32rows(x):
    r, w = x.shape
    return lax.bitcast_convert_type(x.reshape(r, w // 2, 2), jnp.float32)


def _as_bf16rows(x, w):
    return lax.bitcast_convert_type(x, jnp.bfloat16).reshape(x.shape[0], w)

def _routing_indices():
    rkey = jax.random.key(1234)
    ka, kb = jax.random.split(rkey)
    idxs = []
    for key in (ka, kb):
        perm = jax.random.permutation(key, N).astype(jnp.int32)
        j = jnp.arange(N, dtype=jnp.int32)
        t_of_j = (j % TE) * E + j // TE
        src = perm[t_of_j]
        oslot = (perm % E) * TE + perm // E
        idxs.append((src, oslot))
    return idxs


def kernel(inp, w0, w1, w2, fs0, fs2):
    (src0, oslot0), (src2, oslot2) = _routing_indices()
    fs2_inv = jnp.argsort(fs2).astype(jnp.int32)
    idx_shuf0 = (jnp.repeat(jnp.arange(B, dtype=jnp.int32) * F, F)
                 + jnp.tile(fs0.astype(jnp.int32), B))
    qidx = (jnp.arange(E, dtype=jnp.int32)[:, None] * F
            + fs2_inv[None, :]).reshape(-1)
    gcol = (fs2_inv // 192).reshape(1, F)
    w1t = jnp.transpose(w1, (2, 0, 1))

    shuf = _sc_multi_gather([(B * F, S, jnp.float32, 48)])(
        inp.reshape(B * F, S), idx_shuf0)
    xtm = _transpose_to_tokens(shuf.reshape(B, F, S))
    w2p = jnp.pad(w2.reshape(E * F, 192), ((0, 0), (0, 64)))
    xs0, wsm = _sc_multi_gather([(N, F, jnp.float32, 128),
                                 (E * F, 256, jnp.float32, 96)])(
        xtm, w2p, src0, qidx)
    y0_dep, y0_ss = _moe0_matmul(xs0, w0)
    d_dep, d_ss = _sc_multi_gather([(N, F, jnp.float32, 64),
                                    (N, F, jnp.int32, 64)])(
        y0_dep, y0_ss, oslot0, oslot0)
    x2 = _cum_norm_conv_norm(d_dep, d_ss, w1t)
    xs2 = _sc_multi_gather([(N, 384, jnp.int32, 128)])(x2, src2)
    y2 = _moe2_matmul(xs2, wsm.reshape(E, F, 256), gcol)
    out_tok = _sc_multi_gather([(N, 384, jnp.int32, 128)])(y2, oslot2)
    return _transpose_to_features(out_tok)


# speedup vs baseline: 3.6887x; 1.2039x over previous
import functools

import jax
import jax.numpy as jnp
from jax import lax
from jax.experimental import pallas as pl
from jax.experimental.pallas import tpu as pltpu
from jax.experimental.pallas import tpu_sc as plsc

F = 768
S = 2048
B = 2
N = B * S
E = 8
C3 = 2304
TE = N // E
NC = 2
NS = 16
NW = NC * NS


def _sc_multi_gather(tasks):
    mesh = plsc.VectorSubcoreMesh(core_axis_name="c", subcore_axis_name="s")
    per_ws = []
    for n_out, width, dtype, chunk in tasks:
        per_w = n_out // NW
        assert per_w % chunk == 0 and chunk <= 128 and chunk % 8 == 0
        per_ws.append(per_w)

    scratch = []
    for (n_out, width, dtype, chunk), per_w in zip(tasks, per_ws):
        scratch.append(pltpu.VMEM((per_w,), jnp.int32))
        scratch.append(pltpu.VMEM((chunk, width), dtype))
    scratch.append(pltpu.SemaphoreType.DMA)

    @functools.partial(
        pl.kernel,
        out_type=[jax.ShapeDtypeStruct((n_out, width), dtype)
                  for n_out, width, dtype, chunk in tasks],
        mesh=mesh,
        scratch_types=scratch,
    )
    def k(*refs):
        nt = len(tasks)
        tables = refs[:nt]
        idxs = refs[nt:2 * nt]
        outs = refs[2 * nt:3 * nt]
        sv = refs[3 * nt:]
        sem = sv[-1]
        wid = lax.axis_index("s") * NC + lax.axis_index("c")
        for t, (n_out, width, dtype, chunk) in enumerate(tasks):
            per_w = per_ws[t]
            idx_v, buf = sv[2 * t], sv[2 * t + 1]
            base = wid * per_w
            pltpu.sync_copy(idxs[t].at[pl.ds(base, per_w)], idx_v)
            for i in range(per_w // chunk):
                idx_c = idx_v if per_w == chunk else \
                    idx_v.at[pl.ds(i * chunk, chunk)]
                pltpu.async_copy(tables[t].at[idx_c], buf, sem).wait()
                pltpu.sync_copy(
                    buf, outs[t].at[pl.ds(base + i * chunk, chunk)])

    def call(*table_and_idx):
        res = k(*table_and_idx)
        return res if len(tasks) > 1 else res[0]

    return call


def _transpose_to_tokens(x):
    Sb = 256

    def body(x_ref, o_ref):
        o_ref[...] = x_ref[0].T

    return pl.pallas_call(
        body,
        grid=(B, S // Sb),
        in_specs=[pl.BlockSpec((1, F, Sb), lambda b, s: (b, 0, s))],
        out_specs=pl.BlockSpec((Sb, F), lambda b, s: (b * (S // Sb) + s, 0)),
        out_shape=jax.ShapeDtypeStruct((N, F), jnp.float32),
    )(x)


def _transpose_to_features(x):
    Sb = 256

    def body(x_ref, o_ref):
        o_ref[...] = _unpack_half(x_ref[...]).T[None]

    return pl.pallas_call(
        body,
        grid=(B, S // Sb),
        in_specs=[pl.BlockSpec((Sb, 384), lambda b, s: (b * (S // Sb) + s, 0))],
        out_specs=pl.BlockSpec((1, F, Sb), lambda b, s: (b, 0, s)),
        out_shape=jax.ShapeDtypeStruct((B, F, S), jnp.float32),
    )(x)


def _moe0_matmul(xs, w0):
    def body(x_ref, w_ref, dep_ref, ss_ref):
        res = []
        for g in range(4):
            xg = x_ref[:, g * 192:(g + 1) * 192]
            res.append(jnp.dot(
                xg, w_ref[g], preferred_element_type=jnp.float32,
                precision=lax.Precision.DEFAULT))
        dep_ref[:, :576] = res[0]
        dep_ref[:, 576:] = res[1][:, :192]
        ss_ref[:, :384] = _pack16(res[1][:, 192:], res[2][:, :384])
        sh_lo = jnp.concatenate([res[2][:, 384:], res[3][:, :192]], axis=1)
        ss_ref[:, 384:] = _pack16(sh_lo, res[3][:, 192:])

    return pl.pallas_call(
        body,
        grid=(E,),
        in_specs=[pl.BlockSpec((TE, F), lambda e: (e, 0)),
                  pl.BlockSpec((4, 192, 576), lambda e: (e, 0, 0))],
        out_specs=[pl.BlockSpec((TE, F), lambda e: (e, 0)),
                   pl.BlockSpec((TE, F), lambda e: (e, 0))],
        out_shape=[jax.ShapeDtypeStruct((N, F), jnp.float32),
                   jax.ShapeDtypeStruct((N, F), jnp.int32)],
    )(xs, w0)


def _pack16(a, b):
    ai = lax.bitcast_convert_type(a, jnp.int32)
    bi = lax.bitcast_convert_type(b, jnp.int32)
    ra = (ai + 0x7FFF + ((ai >> 16) & 1)) >> 16
    rb = (bi + 0x7FFF + ((bi >> 16) & 1)) >> 16
    return (ra & 0xFFFF) | (rb << 16)


def _unpack16(p):
    a = lax.bitcast_convert_type(p << 16, jnp.float32)
    b = lax.bitcast_convert_type(p & jnp.int32(-65536), jnp.float32)
    return a, b


def _pack_half(x):
    return _pack16(x[:, :384], x[:, 384:])


def _unpack_half(p):
    a, b = _unpack16(p)
    return jnp.concatenate([a, b], axis=1)


def _norm_block(s0, s1, shift):
    s0r = jnp.maximum(s0, 0.0)
    x = s0r * s0r * s0r * s1 + shift
    x = x - jnp.mean(x, axis=1, keepdims=True)
    ssq = jnp.sum(x * x, axis=1, keepdims=True)
    return x * lax.rsqrt(ssq * (1.0 / F))


def _cum_norm_conv_norm(d_dep, d_ss, w1t):
    Rb = 256
    per_batch = S // Rb

    def body(dep_ref, sc_ref, sh_ref, w_ref, o_ref, carry_ref, tail_ref):
        i = pl.program_id(0)

        @pl.when(i % per_batch == 0)
        def _():
            carry_ref[...] = jnp.zeros_like(carry_ref)
            tail_ref[...] = jnp.zeros_like(tail_ref)

        r = lax.broadcasted_iota(jnp.int32, (Rb, Rb), 0)
        c = lax.broadcasted_iota(jnp.int32, (Rb, Rb), 1)
        ltri = (r >= c).astype(jnp.float32)
        cum = jnp.dot(ltri, dep_ref[...], preferred_element_type=jnp.float32,
                      precision=lax.Precision.DEFAULT) + carry_ref[...]
        carry_ref[...] = cum[Rb - 1:Rb, :]
        srow = (i % per_batch) * Rb + lax.broadcasted_iota(
            jnp.int32, (Rb, 1), 0)
        s0 = cum / (srow + 1).astype(jnp.float32)
        x1 = _norm_block(s0, _unpack_half(sc_ref[...]),
                         _unpack_half(sh_ref[...]))

        ext = jnp.concatenate([tail_ref[...], x1], axis=0)
        tail_ref[...] = x1[Rb - 2:, :]
        shifted = [ext[0:Rb], ext[1:Rb + 1], x1]
        cols = []
        for g in range(4):
            acc = None
            for k in range(3):
                xg = shifted[k][:, g * 192:(g + 1) * 192]
                wgk = w_ref[k, g * 576:(g + 1) * 576, :]
                pk = lax.dot_general(
                    xg, wgk, (((1,), (1,)), ((), ())),
                    preferred_element_type=jnp.float32,
                    precision=lax.Precision.DEFAULT)
                acc = pk if acc is None else acc + pk
            cols.append(acc)
        conv = jnp.concatenate(cols, axis=1)
        o_ref[...] = _pack_half(_norm_block(
            conv[:, :768], conv[:, 768:1536], conv[:, 1536:]))

    return pl.pallas_call(
        body,
        grid=(N // Rb,),
        in_specs=[pl.BlockSpec((Rb, F), lambda i: (i, 0)),
                  pl.BlockSpec((Rb, 384), lambda i: (i, 0)),
                  pl.BlockSpec((Rb, 384), lambda i: (i, 1)),
                  pl.BlockSpec((3, C3, 192), lambda i: (0, 0, 0))],
        out_specs=pl.BlockSpec((Rb, 384), lambda i: (i, 0)),
        out_shape=jax.ShapeDtypeStruct((N, 384), jnp.int32),
        scratch_shapes=[pltpu.VMEM((1, F), jnp.float32),
                        pltpu.VMEM((2, F), jnp.float32)],
    )(d_dep, d_ss, d_ss, w1t)


def _moe2_matmul(xs2, wsm, gcol):
    def body(x_ref, w_ref, g_ref, o_ref):
        x = _unpack_half(x_ref[...])
        gc = g_ref[...]
        ys = []
        for g in range(4):
            xg = x * (gc == g).astype(jnp.float32)
            ys.append(jnp.dot(
                xg, w_ref[0, :, :192], preferred_element_type=jnp.float32,
                precision=lax.Precision.DEFAULT))
        y = jnp.concatenate(ys, axis=1)
        o_ref[...] = _pack_half(y)

    return pl.pallas_call(
        body,
        grid=(E,),
        in_specs=[pl.BlockSpec((TE, 384), lambda e: (e, 0)),
                  pl.BlockSpec((1, F, 256), lambda e: (e, 0, 0)),
                  pl.BlockSpec((1, F), lambda e: (0, 0))],
        out_specs=pl.BlockSpec((TE, 384), lambda e: (e, 0)),
        out_shape=jax.ShapeDtypeStruct((N, 384), jnp.int32),
    )(xs2, wsm, gcol)


def _as_f32rows(x):
    r, w = x.shape
    return lax.bitcast_convert_type(x.reshape(r, w // 2, 2), jnp.float32)


def _as_bf16rows(x, w):
    return lax.bitcast_convert_type(x, jnp.bfloat16).reshape(x.shape[0], w)

_ROUTING_CACHE = []


def _routing_indices():
    if not _ROUTING_CACHE:
        import numpy as np
        with jax.ensure_compile_time_eval():
            rkey = jax.random.key(1234)
            ka, kb = jax.random.split(rkey)
            perms = [np.asarray(jax.random.permutation(key, N),
                                dtype=np.int32) for key in (ka, kb)]
        idxs = []
        for perm in perms:
            j = np.arange(N, dtype=np.int32)
            t_of_j = (j % TE) * E + j // TE
            src = perm[t_of_j]
            oslot = (perm % E) * TE + perm // E
            idxs.append((src.astype(np.int32), oslot.astype(np.int32)))
        _ROUTING_CACHE.append(idxs)
    return _ROUTING_CACHE[0]


def kernel(inp, w0, w1, w2, fs0, fs2):
    (src0, oslot0), (src2, oslot2) = _routing_indices()
    fs2_inv = jnp.zeros((F,), jnp.int32).at[fs2].set(
        jnp.arange(F, dtype=jnp.int32))
    idx_shuf0 = (jnp.repeat(jnp.arange(B, dtype=jnp.int32) * F, F)
                 + jnp.tile(fs0.astype(jnp.int32), B))
    qidx = (jnp.arange(E, dtype=jnp.int32)[:, None] * F
            + fs2_inv[None, :]).reshape(-1)
    gcol = (fs2_inv // 192).reshape(1, F)
    w1t = jnp.transpose(w1, (2, 0, 1))

    shuf = _sc_multi_gather([(B * F, S, jnp.float32, 48)])(
        inp.reshape(B * F, S), idx_shuf0)
    xtm = _transpose_to_tokens(shuf.reshape(B, F, S))
    w2p = jnp.pad(w2.reshape(E * F, 192), ((0, 0), (0, 64)))
    xs0, wsm = _sc_multi_gather([(N, F, jnp.float32, 128),
                                 (E * F, 256, jnp.float32, 96)])(
        xtm, w2p, src0, qidx)
    y0_dep, y0_ss = _moe0_matmul(xs0, w0)
    d_dep, d_ss = _sc_multi_gather([(N, F, jnp.float32, 64),
                                    (N, F, jnp.int32, 64)])(
        y0_dep, y0_ss, oslot0, oslot0)
    x2 = _cum_norm_conv_norm(d_dep, d_ss, w1t)
    xs2 = _sc_multi_gather([(N, 384, jnp.int32, 128)])(x2, src2)
    y2 = _moe2_matmul(xs2, wsm.reshape(E, F, 256), gcol)
    out_tok = _sc_multi_gather([(N, 384, jnp.int32, 128)])(y2, oslot2)
    return _transpose_to_features(out_tok)


# speedup vs baseline: 3.7016x; 1.0035x over previous
import functools

import jax
import jax.numpy as jnp
from jax import lax
from jax.experimental import pallas as pl
from jax.experimental.pallas import tpu as pltpu
from jax.experimental.pallas import tpu_sc as plsc

F = 768
S = 2048
B = 2
N = B * S
E = 8
C3 = 2304
TE = N // E
NC = 2
NS = 16
NW = NC * NS


def _sc_multi_gather(tasks):
    mesh = plsc.VectorSubcoreMesh(core_axis_name="c", subcore_axis_name="s")
    per_ws = []
    for n_out, width, dtype, chunk in tasks:
        per_w = n_out // NW
        assert per_w % chunk == 0 and chunk <= 128 and chunk % 8 == 0
        per_ws.append(per_w)

    scratch = []
    for (n_out, width, dtype, chunk), per_w in zip(tasks, per_ws):
        scratch.append(pltpu.VMEM((per_w,), jnp.int32))
        scratch.append(pltpu.VMEM((chunk, width), dtype))
    scratch.append(pltpu.SemaphoreType.DMA)

    @functools.partial(
        pl.kernel,
        out_type=[jax.ShapeDtypeStruct((n_out, width), dtype)
                  for n_out, width, dtype, chunk in tasks],
        mesh=mesh,
        scratch_types=scratch,
    )
    def k(*refs):
        nt = len(tasks)
        tables = refs[:nt]
        idxs = refs[nt:2 * nt]
        outs = refs[2 * nt:3 * nt]
        sv = refs[3 * nt:]
        sem = sv[-1]
        wid = lax.axis_index("s") * NC + lax.axis_index("c")
        for t, (n_out, width, dtype, chunk) in enumerate(tasks):
            per_w = per_ws[t]
            idx_v, buf = sv[2 * t], sv[2 * t + 1]
            base = wid * per_w
            pltpu.sync_copy(idxs[t].at[pl.ds(base, per_w)], idx_v)
            for i in range(per_w // chunk):
                idx_c = idx_v if per_w == chunk else \
                    idx_v.at[pl.ds(i * chunk, chunk)]
                pltpu.async_copy(tables[t].at[idx_c], buf, sem).wait()
                pltpu.sync_copy(
                    buf, outs[t].at[pl.ds(base + i * chunk, chunk)])

    def call(*table_and_idx):
        res = k(*table_and_idx)
        return res if len(tasks) > 1 else res[0]

    return call


def _transpose_to_tokens(x):
    Sb = 256

    def body(x_ref, o_ref):
        o_ref[...] = x_ref[0].T

    return pl.pallas_call(
        body,
        grid=(B, S // Sb),
        in_specs=[pl.BlockSpec((1, F, Sb), lambda b, s: (b, 0, s))],
        out_specs=pl.BlockSpec((Sb, F), lambda b, s: (b * (S // Sb) + s, 0)),
        out_shape=jax.ShapeDtypeStruct((N, F), jnp.float32),
    )(x)


def _transpose_to_features(x):
    Sb = 256

    def body(x_ref, o_ref):
        o_ref[...] = _unpack_half(x_ref[...]).T[None]

    return pl.pallas_call(
        body,
        grid=(B, S // Sb),
        in_specs=[pl.BlockSpec((Sb, 384), lambda b, s: (b * (S // Sb) + s, 0))],
        out_specs=pl.BlockSpec((1, F, Sb), lambda b, s: (b, 0, s)),
        out_shape=jax.ShapeDtypeStruct((B, F, S), jnp.float32),
    )(x)


def _moe0_matmul(xs, w0):
    def body(x_ref, w_ref, dep_ref, ss_ref):
        res = []
        for g in range(4):
            xg = x_ref[:, g * 192:(g + 1) * 192]
            res.append(jnp.dot(
                xg, w_ref[g], preferred_element_type=jnp.float32,
                precision=lax.Precision.DEFAULT))
        dep_ref[:, :576] = res[0]
        dep_ref[:, 576:] = res[1][:, :192]
        ss_ref[:, :384] = _pack16(res[1][:, 192:], res[2][:, :384])
        sh_lo = jnp.concatenate([res[2][:, 384:], res[3][:, :192]], axis=1)
        ss_ref[:, 384:] = _pack16(sh_lo, res[3][:, 192:])

    return pl.pallas_call(
        body,
        grid=(E,),
        in_specs=[pl.BlockSpec((TE, F), lambda e: (e, 0)),
                  pl.BlockSpec((4, 192, 576), lambda e: (e, 0, 0))],
        out_specs=[pl.BlockSpec((TE, F), lambda e: (e, 0)),
                   pl.BlockSpec((TE, F), lambda e: (e, 0))],
        out_shape=[jax.ShapeDtypeStruct((N, F), jnp.float32),
                   jax.ShapeDtypeStruct((N, F), jnp.int32)],
    )(xs, w0)


def _pack16(a, b):
    ai = lax.bitcast_convert_type(a, jnp.int32)
    bi = lax.bitcast_convert_type(b, jnp.int32)
    ra = (ai + 0x7FFF + ((ai >> 16) & 1)) >> 16
    rb = (bi + 0x7FFF + ((bi >> 16) & 1)) >> 16
    return (ra & 0xFFFF) | (rb << 16)


def _unpack16(p):
    a = lax.bitcast_convert_type(p << 16, jnp.float32)
    b = lax.bitcast_convert_type(p & jnp.int32(-65536), jnp.float32)
    return a, b


def _pack_half(x):
    return _pack16(x[:, :384], x[:, 384:])


def _unpack_half(p):
    a, b = _unpack16(p)
    return jnp.concatenate([a, b], axis=1)


def _norm_block(s0, s1, shift):
    s0r = jnp.maximum(s0, 0.0)
    x = s0r * s0r * s0r * s1 + shift
    x = x - jnp.mean(x, axis=1, keepdims=True)
    ssq = jnp.sum(x * x, axis=1, keepdims=True)
    return x * lax.rsqrt(ssq * (1.0 / F))


def _cum_norm_conv_norm(d_dep, d_ss, w1t):
    Rb = 512
    per_batch = S // Rb

    def body(dep_ref, sc_ref, sh_ref, w_ref, o_ref, carry_ref, tail_ref):
        i = pl.program_id(0)

        @pl.when(i % per_batch == 0)
        def _():
            carry_ref[...] = jnp.zeros_like(carry_ref)
            tail_ref[...] = jnp.zeros_like(tail_ref)

        r = lax.broadcasted_iota(jnp.int32, (Rb, Rb), 0)
        c = lax.broadcasted_iota(jnp.int32, (Rb, Rb), 1)
        ltri = (r >= c).astype(jnp.float32)
        cum = jnp.dot(ltri, dep_ref[...], preferred_element_type=jnp.float32,
                      precision=lax.Precision.DEFAULT) + carry_ref[...]
        carry_ref[...] = cum[Rb - 1:Rb, :]
        srow = (i % per_batch) * Rb + lax.broadcasted_iota(
            jnp.int32, (Rb, 1), 0)
        s0 = cum / (srow + 1).astype(jnp.float32)
        x1 = _norm_block(s0, _unpack_half(sc_ref[...]),
                         _unpack_half(sh_ref[...]))

        ext = jnp.concatenate([tail_ref[...], x1], axis=0)
        tail_ref[...] = x1[Rb - 2:, :]
        shifted = [ext[0:Rb], ext[1:Rb + 1], x1]
        cols = []
        for g in range(4):
            acc = None
            for k in range(3):
                xg = shifted[k][:, g * 192:(g + 1) * 192]
                wgk = w_ref[k, g * 576:(g + 1) * 576, :]
                pk = lax.dot_general(
                    xg, wgk, (((1,), (1,)), ((), ())),
                    preferred_element_type=jnp.float32,
                    precision=lax.Precision.DEFAULT)
                acc = pk if acc is None else acc + pk
            cols.append(acc)
        conv = jnp.concatenate(cols, axis=1)
        o_ref[...] = _pack_half(_norm_block(
            conv[:, :768], conv[:, 768:1536], conv[:, 1536:]))

    return pl.pallas_call(
        body,
        grid=(N // Rb,),
        in_specs=[pl.BlockSpec((Rb, F), lambda i: (i, 0)),
                  pl.BlockSpec((Rb, 384), lambda i: (i, 0)),
                  pl.BlockSpec((Rb, 384), lambda i: (i, 1)),
                  pl.BlockSpec((3, C3, 192), lambda i: (0, 0, 0))],
        out_specs=pl.BlockSpec((Rb, 384), lambda i: (i, 0)),
        out_shape=jax.ShapeDtypeStruct((N, 384), jnp.int32),
        scratch_shapes=[pltpu.VMEM((1, F), jnp.float32),
                        pltpu.VMEM((2, F), jnp.float32)],
    )(d_dep, d_ss, d_ss, w1t)


def _moe2_matmul(xs2, wsm, gcol):
    def body(x_ref, w_ref, g_ref, o_ref):
        x = _unpack_half(x_ref[...])
        gc = g_ref[...]
        ys = []
        for g in range(4):
            xg = x * (gc == g).astype(jnp.float32)
            ys.append(jnp.dot(
                xg, w_ref[0, :, :192], preferred_element_type=jnp.float32,
                precision=lax.Precision.DEFAULT))
        y = jnp.concatenate(ys, axis=1)
        o_ref[...] = _pack_half(y)

    return pl.pallas_call(
        body,
        grid=(E,),
        in_specs=[pl.BlockSpec((TE, 384), lambda e: (e, 0)),
                  pl.BlockSpec((1, F, 256), lambda e: (e, 0, 0)),
                  pl.BlockSpec((1, F), lambda e: (0, 0))],
        out_specs=pl.BlockSpec((TE, 384), lambda e: (e, 0)),
        out_shape=jax.ShapeDtypeStruct((N, 384), jnp.int32),
    )(xs2, wsm, gcol)


def _as_f32rows(x):
    r, w = x.shape
    return lax.bitcast_convert_type(x.reshape(r, w // 2, 2), jnp.float32)


def _as_bf16rows(x, w):
    return lax.bitcast_convert_type(x, jnp.bfloat16).reshape(x.shape[0], w)

_ROUTING_CACHE = []


def _routing_indices():
    if not _ROUTING_CACHE:
        import numpy as np
        with jax.ensure_compile_time_eval():
            rkey = jax.random.key(1234)
            ka, kb = jax.random.split(rkey)
            perms = [np.asarray(jax.random.permutation(key, N),
                                dtype=np.int32) for key in (ka, kb)]
        idxs = []
        for perm in perms:
            j = np.arange(N, dtype=np.int32)
            t_of_j = (j % TE) * E + j // TE
            src = perm[t_of_j]
            oslot = (perm % E) * TE + perm // E
            idxs.append((src.astype(np.int32), oslot.astype(np.int32)))
        _ROUTING_CACHE.append(idxs)
    return _ROUTING_CACHE[0]


def kernel(inp, w0, w1, w2, fs0, fs2):
    (src0, oslot0), (src2, oslot2) = _routing_indices()
    fs2_inv = jnp.zeros((F,), jnp.int32).at[fs2].set(
        jnp.arange(F, dtype=jnp.int32))
    idx_shuf0 = (jnp.repeat(jnp.arange(B, dtype=jnp.int32) * F, F)
                 + jnp.tile(fs0.astype(jnp.int32), B))
    qidx = (jnp.arange(E, dtype=jnp.int32)[:, None] * F
            + fs2_inv[None, :]).reshape(-1)
    gcol = (fs2_inv // 192).reshape(1, F)
    w1t = jnp.transpose(w1, (2, 0, 1))

    shuf = _sc_multi_gather([(B * F, S, jnp.float32, 48)])(
        inp.reshape(B * F, S), idx_shuf0)
    xtm = _transpose_to_tokens(shuf.reshape(B, F, S))
    w2p = jnp.pad(w2.reshape(E * F, 192), ((0, 0), (0, 64)))
    xs0, wsm = _sc_multi_gather([(N, F, jnp.float32, 128),
                                 (E * F, 256, jnp.float32, 96)])(
        xtm, w2p, src0, qidx)
    y0_dep, y0_ss = _moe0_matmul(xs0, w0)
    d_dep, d_ss = _sc_multi_gather([(N, F, jnp.float32, 64),
                                    (N, F, jnp.int32, 64)])(
        y0_dep, y0_ss, oslot0, oslot0)
    x2 = _cum_norm_conv_norm(d_dep, d_ss, w1t)
    xs2 = _sc_multi_gather([(N, 384, jnp.int32, 128)])(x2, src2)
    y2 = _moe2_matmul(xs2, wsm.reshape(E, F, 256), gcol)
    out_tok = _sc_multi_gather([(N, 384, jnp.int32, 128)])(y2, oslot2)
    return _transpose_to_features(out_tok)


# speedup vs baseline: 3.8645x; 1.0440x over previous
import functools

import jax
import jax.numpy as jnp
from jax import lax
from jax.experimental import pallas as pl
from jax.experimental.pallas import tpu as pltpu
from jax.experimental.pallas import tpu_sc as plsc

F = 768
S = 2048
B = 2
N = B * S
E = 8
C3 = 2304
TE = N // E
NC = 2
NS = 16
NW = NC * NS


def _sc_multi_gather(tasks):
    mesh = plsc.VectorSubcoreMesh(core_axis_name="c", subcore_axis_name="s")
    per_ws = []
    for n_out, width, dtype, chunk in tasks:
        per_w = n_out // NW
        assert per_w % chunk == 0 and chunk <= 128 and chunk % 8 == 0
        per_ws.append(per_w)

    scratch = []
    for (n_out, width, dtype, chunk), per_w in zip(tasks, per_ws):
        scratch.append(pltpu.VMEM((per_w,), jnp.int32))
        scratch.append(pltpu.VMEM((chunk, width), dtype))
    scratch.append(pltpu.SemaphoreType.DMA)

    @functools.partial(
        pl.kernel,
        out_type=[jax.ShapeDtypeStruct((n_out, width), dtype)
                  for n_out, width, dtype, chunk in tasks],
        mesh=mesh,
        scratch_types=scratch,
    )
    def k(*refs):
        nt = len(tasks)
        tables = refs[:nt]
        idxs = refs[nt:2 * nt]
        outs = refs[2 * nt:3 * nt]
        sv = refs[3 * nt:]
        sem = sv[-1]
        wid = lax.axis_index("s") * NC + lax.axis_index("c")
        for t, (n_out, width, dtype, chunk) in enumerate(tasks):
            per_w = per_ws[t]
            idx_v, buf = sv[2 * t], sv[2 * t + 1]
            base = wid * per_w
            pltpu.sync_copy(idxs[t].at[pl.ds(base, per_w)], idx_v)
            for i in range(per_w // chunk):
                idx_c = idx_v if per_w == chunk else \
                    idx_v.at[pl.ds(i * chunk, chunk)]
                pltpu.async_copy(tables[t].at[idx_c], buf, sem).wait()
                pltpu.sync_copy(
                    buf, outs[t].at[pl.ds(base + i * chunk, chunk)])

    def call(*table_and_idx):
        res = k(*table_and_idx)
        return res if len(tasks) > 1 else res[0]

    return call


def _transpose_to_tokens(x):
    Sb = 512

    def body(x_ref, o_ref):
        o_ref[...] = x_ref[0].T

    return pl.pallas_call(
        body,
        grid=(B, S // Sb),
        in_specs=[pl.BlockSpec((1, F, Sb), lambda b, s: (b, 0, s))],
        out_specs=pl.BlockSpec((Sb, F), lambda b, s: (b * (S // Sb) + s, 0)),
        out_shape=jax.ShapeDtypeStruct((N, F), jnp.float32),
    )(x)


def _transpose_to_features(x):
    Sb = 512

    def body(x_ref, o_ref):
        o_ref[...] = _unpack_half(x_ref[...]).T[None]

    return pl.pallas_call(
        body,
        grid=(B, S // Sb),
        in_specs=[pl.BlockSpec((Sb, 384), lambda b, s: (b * (S // Sb) + s, 0))],
        out_specs=pl.BlockSpec((1, F, Sb), lambda b, s: (b, 0, s)),
        out_shape=jax.ShapeDtypeStruct((B, F, S), jnp.float32),
    )(x)


def _moe0_matmul(xs, w0):
    def body(x_ref, w_ref, dep_ref, ss_ref):
        res = []
        for g in range(4):
            xg = x_ref[:, g * 192:(g + 1) * 192]
            res.append(jnp.dot(
                xg, w_ref[g], preferred_element_type=jnp.float32,
                precision=lax.Precision.DEFAULT))
        dep_ref[:, :576] = res[0]
        dep_ref[:, 576:] = res[1][:, :192]
        ss_ref[:, :384] = _pack16(res[1][:, 192:], res[2][:, :384])
        sh_lo = jnp.concatenate([res[2][:, 384:], res[3][:, :192]], axis=1)
        ss_ref[:, 384:] = _pack16(sh_lo, res[3][:, 192:])

    return pl.pallas_call(
        body,
        grid=(E,),
        in_specs=[pl.BlockSpec((TE, F), lambda e: (e, 0)),
                  pl.BlockSpec((4, 192, 576), lambda e: (e, 0, 0))],
        out_specs=[pl.BlockSpec((TE, F), lambda e: (e, 0)),
                   pl.BlockSpec((TE, F), lambda e: (e, 0))],
        out_shape=[jax.ShapeDtypeStruct((N, F), jnp.float32),
                   jax.ShapeDtypeStruct((N, F), jnp.int32)],
    )(xs, w0)


def _pack16(a, b):
    ai = lax.bitcast_convert_type(a, jnp.int32)
    bi = lax.bitcast_convert_type(b, jnp.int32)
    ra = (ai + 0x7FFF + ((ai >> 16) & 1)) >> 16
    rb = (bi + 0x7FFF + ((bi >> 16) & 1)) >> 16
    return (ra & 0xFFFF) | (rb << 16)


def _unpack16(p):
    a = lax.bitcast_convert_type(p << 16, jnp.float32)
    b = lax.bitcast_convert_type(p & jnp.int32(-65536), jnp.float32)
    return a, b


def _pack_half(x):
    return _pack16(x[:, :384], x[:, 384:])


def _unpack_half(p):
    a, b = _unpack16(p)
    return jnp.concatenate([a, b], axis=1)


def _norm_block(s0, s1, shift):
    s0r = jnp.maximum(s0, 0.0)
    x = s0r * s0r * s0r * s1 + shift
    x = x - jnp.mean(x, axis=1, keepdims=True)
    ssq = jnp.sum(x * x, axis=1, keepdims=True)
    return x * lax.rsqrt(ssq * (1.0 / F))


def _cum_norm_conv_norm(d_dep, d_ss, w1t):
    Rb = 512
    per_batch = S // Rb

    def body(dep_ref, sc_ref, sh_ref, w_ref, o_ref, carry_ref, tail_ref):
        i = pl.program_id(0)

        @pl.when(i % per_batch == 0)
        def _():
            carry_ref[...] = jnp.zeros_like(carry_ref)
            tail_ref[...] = jnp.zeros_like(tail_ref)

        r = lax.broadcasted_iota(jnp.int32, (Rb, Rb), 0)
        c = lax.broadcasted_iota(jnp.int32, (Rb, Rb), 1)
        ltri = (r >= c).astype(jnp.float32)
        cum = jnp.dot(ltri, dep_ref[...], preferred_element_type=jnp.float32,
                      precision=lax.Precision.DEFAULT) + carry_ref[...]
        carry_ref[...] = cum[Rb - 1:Rb, :]
        srow = (i % per_batch) * Rb + lax.broadcasted_iota(
            jnp.int32, (Rb, 1), 0)
        s0 = cum / (srow + 1).astype(jnp.float32)
        x1 = _norm_block(s0, _unpack_half(sc_ref[...]),
                         _unpack_half(sh_ref[...]))

        ext = jnp.concatenate([tail_ref[...], x1], axis=0)
        tail_ref[...] = x1[Rb - 2:, :]
        shifted = [ext[0:Rb], ext[1:Rb + 1], x1]
        cols = []
        for g in range(4):
            acc = None
            for k in range(3):
                xg = shifted[k][:, g * 192:(g + 1) * 192]
                wgk = w_ref[k, g * 576:(g + 1) * 576, :]
                pk = lax.dot_general(
                    xg, wgk, (((1,), (1,)), ((), ())),
                    preferred_element_type=jnp.float32,
                    precision=lax.Precision.DEFAULT)
                acc = pk if acc is None else acc + pk
            cols.append(acc)
        conv = jnp.concatenate(cols, axis=1)
        o_ref[...] = _pack_half(_norm_block(
            conv[:, :768], conv[:, 768:1536], conv[:, 1536:]))

    return pl.pallas_call(
        body,
        grid=(N // Rb,),
        in_specs=[pl.BlockSpec((Rb, F), lambda i: (i, 0)),
                  pl.BlockSpec((Rb, 384), lambda i: (i, 0)),
                  pl.BlockSpec((Rb, 384), lambda i: (i, 1)),
                  pl.BlockSpec((3, C3, 192), lambda i: (0, 0, 0))],
        out_specs=pl.BlockSpec((Rb, 384), lambda i: (i, 0)),
        out_shape=jax.ShapeDtypeStruct((N, 384), jnp.int32),
        scratch_shapes=[pltpu.VMEM((1, F), jnp.float32),
                        pltpu.VMEM((2, F), jnp.float32)],
    )(d_dep, d_ss, d_ss, w1t)


def _moe2_matmul(xs2, wsm, gcol):
    def body(x_ref, w_ref, g_ref, o_ref):
        x = _unpack_half(x_ref[...])
        gc = g_ref[...]
        ys = []
        for g in range(4):
            xg = x * (gc == g).astype(jnp.float32)
            ys.append(jnp.dot(
                xg, w_ref[0, :, :192], preferred_element_type=jnp.float32,
                precision=lax.Precision.DEFAULT))
        y = jnp.concatenate(ys, axis=1)
        o_ref[...] = _pack_half(y)

    return pl.pallas_call(
        body,
        grid=(E,),
        in_specs=[pl.BlockSpec((TE, 384), lambda e: (e, 0)),
                  pl.BlockSpec((1, F, 256), lambda e: (e, 0, 0)),
                  pl.BlockSpec((1, F), lambda e: (0, 0))],
        out_specs=pl.BlockSpec((TE, 384), lambda e: (e, 0)),
        out_shape=jax.ShapeDtypeStruct((N, 384), jnp.int32),
    )(xs2, wsm, gcol)


def _as_f32rows(x):
    r, w = x.shape
    return lax.bitcast_convert_type(x.reshape(r, w // 2, 2), jnp.float32)


def _as_bf16rows(x, w):
    return lax.bitcast_convert_type(x, jnp.bfloat16).reshape(x.shape[0], w)

_ROUTING_CACHE = []


def _routing_indices():
    if not _ROUTING_CACHE:
        import numpy as np
        with jax.ensure_compile_time_eval():
            rkey = jax.random.key(1234)
            ka, kb = jax.random.split(rkey)
            perms = [np.asarray(jax.random.permutation(key, N),
                                dtype=np.int32) for key in (ka, kb)]
        idxs = []
        for perm in perms:
            j = np.arange(N, dtype=np.int32)
            t_of_j = (j % TE) * E + j // TE
            src = perm[t_of_j]
            oslot = (perm % E) * TE + perm // E
            idxs.append((src.astype(np.int32), oslot.astype(np.int32)))
        _ROUTING_CACHE.append(idxs)
    return _ROUTING_CACHE[0]


def kernel(inp, w0, w1, w2, fs0, fs2):
    (src0, oslot0), (src2, oslot2) = _routing_indices()
    fs2_inv = jnp.zeros((F,), jnp.int32).at[fs2].set(
        jnp.arange(F, dtype=jnp.int32))
    idx_shuf0 = (jnp.repeat(jnp.arange(B, dtype=jnp.int32) * F, F)
                 + jnp.tile(fs0.astype(jnp.int32), B))
    qidx = (jnp.arange(E, dtype=jnp.int32)[:, None] * F
            + fs2_inv[None, :]).reshape(-1)
    gcol = (fs2_inv // 192).reshape(1, F)
    w1t = jnp.transpose(w1, (2, 0, 1))

    shuf = _sc_multi_gather([(B * F, S, jnp.float32, 48)])(
        inp.reshape(B * F, S), idx_shuf0)
    xtm = _transpose_to_tokens(shuf.reshape(B, F, S))
    w2p = jnp.pad(w2.reshape(E * F, 192), ((0, 0), (0, 64)))
    xs0, wsm = _sc_multi_gather([(N, F, jnp.float32, 128),
                                 (E * F, 256, jnp.float32, 96)])(
        xtm, w2p, src0, qidx)
    y0_dep, y0_ss = _moe0_matmul(xs0, w0)
    d_dep, d_ss = _sc_multi_gather([(N, F, jnp.float32, 64),
                                    (N, F, jnp.int32, 64)])(
        y0_dep, y0_ss, oslot0, oslot0)
    x2 = _cum_norm_conv_norm(d_dep, d_ss, w1t)
    xs2 = _sc_multi_gather([(N, 384, jnp.int32, 128)])(x2, src2)
    y2 = _moe2_matmul(xs2, wsm.reshape(E, F, 256), gcol)
    out_tok = _sc_multi_gather([(N, 384, jnp.int32, 128)])(y2, oslot2)
    return _transpose_to_features(out_tok)


# speedup vs baseline: 3.9021x; 1.0097x over previous
import functools

import jax
import jax.numpy as jnp
from jax import lax
from jax.experimental import pallas as pl
from jax.experimental.pallas import tpu as pltpu
from jax.experimental.pallas import tpu_sc as plsc

F = 768
S = 2048
B = 2
N = B * S
E = 8
C3 = 2304
TE = N // E
NC = 2
NS = 16
NW = NC * NS


def _sc_multi_gather(tasks):
    mesh = plsc.VectorSubcoreMesh(core_axis_name="c", subcore_axis_name="s")
    per_ws = []
    for n_out, width, dtype, chunk in tasks:
        per_w = n_out // NW
        assert per_w % chunk == 0 and chunk <= 128 and chunk % 8 == 0
        per_ws.append(per_w)

    scratch = []
    for (n_out, width, dtype, chunk), per_w in zip(tasks, per_ws):
        scratch.append(pltpu.VMEM((per_w,), jnp.int32))
        scratch.append(pltpu.VMEM((chunk, width), dtype))
    scratch.append(pltpu.SemaphoreType.DMA)

    @functools.partial(
        pl.kernel,
        out_type=[jax.ShapeDtypeStruct((n_out, width), dtype)
                  for n_out, width, dtype, chunk in tasks],
        mesh=mesh,
        scratch_types=scratch,
    )
    def k(*refs):
        nt = len(tasks)
        tables = refs[:nt]
        idxs = refs[nt:2 * nt]
        outs = refs[2 * nt:3 * nt]
        sv = refs[3 * nt:]
        sem = sv[-1]
        wid = lax.axis_index("s") * NC + lax.axis_index("c")
        for t, (n_out, width, dtype, chunk) in enumerate(tasks):
            per_w = per_ws[t]
            idx_v, buf = sv[2 * t], sv[2 * t + 1]
            base = wid * per_w
            pltpu.sync_copy(idxs[t].at[pl.ds(base, per_w)], idx_v)
            for i in range(per_w // chunk):
                idx_c = idx_v if per_w == chunk else \
                    idx_v.at[pl.ds(i * chunk, chunk)]
                pltpu.async_copy(tables[t].at[idx_c], buf, sem).wait()
                pltpu.sync_copy(
                    buf, outs[t].at[pl.ds(base + i * chunk, chunk)])

    def call(*table_and_idx):
        res = k(*table_and_idx)
        return res if len(tasks) > 1 else res[0]

    return call


def _transpose_to_tokens(x):
    Sb = 512

    def body(x_ref, o_ref):
        o_ref[...] = x_ref[0].T

    return pl.pallas_call(
        body,
        grid=(B, S // Sb),
        in_specs=[pl.BlockSpec((1, F, Sb), lambda b, s: (b, 0, s))],
        out_specs=pl.BlockSpec((Sb, F), lambda b, s: (b * (S // Sb) + s, 0)),
        out_shape=jax.ShapeDtypeStruct((N, F), jnp.float32),
    )(x)


def _transpose_to_features(x):
    Sb = 512

    def body(x_ref, o_ref):
        o_ref[...] = _unpack_half(x_ref[...]).T[None]

    return pl.pallas_call(
        body,
        grid=(B, S // Sb),
        in_specs=[pl.BlockSpec((Sb, 384), lambda b, s: (b * (S // Sb) + s, 0))],
        out_specs=pl.BlockSpec((1, F, Sb), lambda b, s: (b, 0, s)),
        out_shape=jax.ShapeDtypeStruct((B, F, S), jnp.float32),
    )(x)


def _moe0_matmul(xs, w0):
    def body(x_ref, w_ref, o_ref):
        res = []
        for g in range(4):
            xg = x_ref[:, g * 192:(g + 1) * 192]
            res.append(jnp.dot(
                xg, w_ref[g], preferred_element_type=jnp.float32,
                precision=lax.Precision.DEFAULT))
        o_ref[:, :576] = lax.bitcast_convert_type(res[0], jnp.int32)
        o_ref[:, 576:768] = lax.bitcast_convert_type(res[1][:, :192],
                                                     jnp.int32)
        o_ref[:, 768:1152] = _pack16(res[1][:, 192:], res[2][:, :384])
        sh_lo = jnp.concatenate([res[2][:, 384:], res[3][:, :192]], axis=1)
        o_ref[:, 1152:] = _pack16(sh_lo, res[3][:, 192:])

    return pl.pallas_call(
        body,
        grid=(E,),
        in_specs=[pl.BlockSpec((TE, F), lambda e: (e, 0)),
                  pl.BlockSpec((4, 192, 576), lambda e: (e, 0, 0))],
        out_specs=pl.BlockSpec((TE, 1536), lambda e: (e, 0)),
        out_shape=jax.ShapeDtypeStruct((N, 1536), jnp.int32),
    )(xs, w0)


def _pack16(a, b):
    ai = lax.bitcast_convert_type(a, jnp.int32)
    bi = lax.bitcast_convert_type(b, jnp.int32)
    ra = (ai + 0x7FFF + ((ai >> 16) & 1)) >> 16
    rb = (bi + 0x7FFF + ((bi >> 16) & 1)) >> 16
    return (ra & 0xFFFF) | (rb << 16)


def _unpack16(p):
    a = lax.bitcast_convert_type(p << 16, jnp.float32)
    b = lax.bitcast_convert_type(p & jnp.int32(-65536), jnp.float32)
    return a, b


def _pack_half(x):
    return _pack16(x[:, :384], x[:, 384:])


def _unpack_half(p):
    a, b = _unpack16(p)
    return jnp.concatenate([a, b], axis=1)


def _norm_block(s0, s1, shift):
    s0r = jnp.maximum(s0, 0.0)
    x = s0r * s0r * s0r * s1 + shift
    x = x - jnp.mean(x, axis=1, keepdims=True)
    ssq = jnp.sum(x * x, axis=1, keepdims=True)
    return x * lax.rsqrt(ssq * (1.0 / F))


def _cum_norm_conv_norm(d_all, w1t):
    Rb = 512
    per_batch = S // Rb

    def body(dep_ref, sc_ref, sh_ref, w_ref, o_ref, carry_ref, tail_ref):
        i = pl.program_id(0)

        @pl.when(i % per_batch == 0)
        def _():
            carry_ref[...] = jnp.zeros_like(carry_ref)
            tail_ref[...] = jnp.zeros_like(tail_ref)

        r = lax.broadcasted_iota(jnp.int32, (Rb, Rb), 0)
        c = lax.broadcasted_iota(jnp.int32, (Rb, Rb), 1)
        ltri = (r >= c).astype(jnp.float32)
        dep = lax.bitcast_convert_type(dep_ref[...], jnp.float32)
        cum = jnp.dot(ltri, dep, preferred_element_type=jnp.float32,
                      precision=lax.Precision.DEFAULT) + carry_ref[...]
        carry_ref[...] = cum[Rb - 1:Rb, :]
        srow = (i % per_batch) * Rb + lax.broadcasted_iota(
            jnp.int32, (Rb, 1), 0)
        s0 = cum / (srow + 1).astype(jnp.float32)
        x1 = _norm_block(s0, _unpack_half(sc_ref[...]),
                         _unpack_half(sh_ref[...]))

        ext = jnp.concatenate([tail_ref[...], x1], axis=0)
        tail_ref[...] = x1[Rb - 2:, :]
        shifted = [ext[0:Rb], ext[1:Rb + 1], x1]
        cols = []
        for g in range(4):
            acc = None
            for k in range(3):
                xg = shifted[k][:, g * 192:(g + 1) * 192]
                wgk = w_ref[k, g * 576:(g + 1) * 576, :]
                pk = lax.dot_general(
                    xg, wgk, (((1,), (1,)), ((), ())),
                    preferred_element_type=jnp.float32,
                    precision=lax.Precision.DEFAULT)
                acc = pk if acc is None else acc + pk
            cols.append(acc)
        conv = jnp.concatenate(cols, axis=1)
        o_ref[...] = _pack_half(_norm_block(
            conv[:, :768], conv[:, 768:1536], conv[:, 1536:]))

    return pl.pallas_call(
        body,
        grid=(N // Rb,),
        in_specs=[pl.BlockSpec((Rb, F), lambda i: (i, 0)),
                  pl.BlockSpec((Rb, 384), lambda i: (i, 2)),
                  pl.BlockSpec((Rb, 384), lambda i: (i, 3)),
                  pl.BlockSpec((3, C3, 192), lambda i: (0, 0, 0))],
        out_specs=pl.BlockSpec((Rb, 384), lambda i: (i, 0)),
        out_shape=jax.ShapeDtypeStruct((N, 384), jnp.int32),
        scratch_shapes=[pltpu.VMEM((1, F), jnp.float32),
                        pltpu.VMEM((2, F), jnp.float32)],
    )(d_all, d_all, d_all, w1t)


def _moe2_matmul(xs2, wsm, gcol):
    def body(x_ref, w_ref, g_ref, o_ref):
        x = _unpack_half(x_ref[...])
        gc = g_ref[...]
        ys = []
        for g in range(4):
            xg = x * (gc == g).astype(jnp.float32)
            ys.append(jnp.dot(
                xg, w_ref[0, :, :192], preferred_element_type=jnp.float32,
                precision=lax.Precision.DEFAULT))
        y = jnp.concatenate(ys, axis=1)
        o_ref[...] = _pack_half(y)

    return pl.pallas_call(
        body,
        grid=(E,),
        in_specs=[pl.BlockSpec((TE, 384), lambda e: (e, 0)),
                  pl.BlockSpec((1, F, 256), lambda e: (e, 0, 0)),
                  pl.BlockSpec((1, F), lambda e: (0, 0))],
        out_specs=pl.BlockSpec((TE, 384), lambda e: (e, 0)),
        out_shape=jax.ShapeDtypeStruct((N, 384), jnp.int32),
    )(xs2, wsm, gcol)


def _as_f32rows(x):
    r, w = x.shape
    return lax.bitcast_convert_type(x.reshape(r, w // 2, 2), jnp.float32)


def _as_bf16rows(x, w):
    return lax.bitcast_convert_type(x, jnp.bfloat16).reshape(x.shape[0], w)

_ROUTING_CACHE = []


def _routing_indices():
    if not _ROUTING_CACHE:
        import numpy as np
        with jax.ensure_compile_time_eval():
            rkey = jax.random.key(1234)
            ka, kb = jax.random.split(rkey)
            perms = [np.asarray(jax.random.permutation(key, N),
                                dtype=np.int32) for key in (ka, kb)]
        idxs = []
        for perm in perms:
            j = np.arange(N, dtype=np.int32)
            t_of_j = (j % TE) * E + j // TE
            src = perm[t_of_j]
            oslot = (perm % E) * TE + perm // E
            idxs.append((src.astype(np.int32), oslot.astype(np.int32)))
        _ROUTING_CACHE.append(idxs)
    return _ROUTING_CACHE[0]


def kernel(inp, w0, w1, w2, fs0, fs2):
    (src0, oslot0), (src2, oslot2) = _routing_indices()
    fs2_inv = jnp.zeros((F,), jnp.int32).at[fs2].set(
        jnp.arange(F, dtype=jnp.int32))
    idx_shuf0 = (jnp.repeat(jnp.arange(B, dtype=jnp.int32) * F, F)
                 + jnp.tile(fs0.astype(jnp.int32), B))
    qidx = (jnp.arange(E, dtype=jnp.int32)[:, None] * F
            + fs2_inv[None, :]).reshape(-1)
    gcol = (fs2_inv // 192).reshape(1, F)
    w1t = jnp.transpose(w1, (2, 0, 1))

    shuf = _sc_multi_gather([(B * F, S, jnp.float32, 48)])(
        inp.reshape(B * F, S), idx_shuf0)
    xtm = _transpose_to_tokens(shuf.reshape(B, F, S))
    w2p = jnp.pad(w2.reshape(E * F, 192), ((0, 0), (0, 64)))
    xs0, wsm = _sc_multi_gather([(N, F, jnp.float32, 128),
                                 (E * F, 256, jnp.float32, 96)])(
        xtm, w2p, src0, qidx)
    y0 = _moe0_matmul(xs0, w0)
    d_all = _sc_multi_gather([(N, 1536, jnp.int32, 64)])(y0, oslot0)
    x2 = _cum_norm_conv_norm(d_all, w1t)
    xs2 = _sc_multi_gather([(N, 384, jnp.int32, 128)])(x2, src2)
    y2 = _moe2_matmul(xs2, wsm.reshape(E, F, 256), gcol)
    out_tok = _sc_multi_gather([(N, 384, jnp.int32, 128)])(y2, oslot2)
    return _transpose_to_features(out_tok)
